# baseline (speedup 1.0000x reference)
import functools

import jax
import jax.numpy as jnp
from jax import lax
from jax.experimental import pallas as pl
from jax.experimental.pallas import tpu as pltpu
from jax.experimental.pallas import tpu_sc as plsc

N = 10000
E = 320000
D = 128
DE = 16

NC = 2
NS = 16
NW = NC * NS
EPT = E // NW
K = 128
CH = 78
KT = EPT - CH * K
N_PAD = 10112
ROWS_PER_TILE = N_PAD // NS

Q = 4
EPT_B = 10240
SB_FULL = EPT_B // (Q * K)
SB_LAST = (E - 31 * EPT_B) // (Q * K)


def _sc_agg(x, edge_index, zeros_big):
    mesh = plsc.VectorSubcoreMesh(core_axis_name="c", subcore_axis_name="s")

    @functools.partial(
        pl.kernel,
        out_type=jax.ShapeDtypeStruct((NC, N_PAD, D), jnp.float32),
        mesh=mesh,
        scratch_types=[
            pltpu.VMEM((K,), jnp.int32),
            pltpu.VMEM((K,), jnp.int32),
            pltpu.VMEM((K,), jnp.int32),
            pltpu.VMEM((K,), jnp.int32),
            pltpu.VMEM((KT,), jnp.int32),
            pltpu.VMEM((KT,), jnp.int32),
            pltpu.VMEM((K, D), jnp.float32),
            pltpu.VMEM((K, D), jnp.float32),
            pltpu.VMEM_SHARED((N_PAD, D), jnp.float32),
            pltpu.SemaphoreType.DMA,
            pltpu.SemaphoreType.DMA,
            pltpu.SemaphoreType.DMA,
            pltpu.SemaphoreType.DMA,
            pltpu.SemaphoreType.DMA,
            pltpu.SemaphoreType.DMA,
        ],
        compiler_params=pltpu.CompilerParams(use_tc_tiling_on_sc=False),
    )
    def sc_body(x_hbm, ei_hbm, z_hbm,
                agg_out, srcv0, srcv1, dstv0, dstv1, srct, dstt,
                rows0, rows1, agg_sh, si0, si1, sg0, sg1, ss0, ss1):
        cid = lax.axis_index("c")
        sid = lax.axis_index("s")
        wid = cid * NS + sid
        base = wid * EPT
        srcv = (srcv0, srcv1)
        dstv = (dstv0, dstv1)
        rows = (rows0, rows1)
        si = (si0, si1)
        sg = (sg0, sg1)
        ss = (ss0, ss1)

        r0 = sid * ROWS_PER_TILE
        pltpu.sync_copy(z_hbm, rows0)
        for j, sz in ((0, K), (1, K), (2, K), (3, K), (4, ROWS_PER_TILE - 4 * K)):
            pltpu.sync_copy(rows0.at[pl.ds(0, sz)],
                            agg_sh.at[pl.ds(r0 + j * K, sz)])
        plsc.subcore_barrier()

        def idx_start(i, b):
            pltpu.async_copy(ei_hbm.at[0, pl.ds(base + i * K, K)], srcv[b], si[b])
            pltpu.async_copy(ei_hbm.at[1, pl.ds(base + i * K, K)], dstv[b], si[b])

        def idx_wait(i, b):
            pltpu.make_async_copy(ei_hbm.at[0, pl.ds(base + i * K, K)],
                                  srcv[b], si[b]).wait()
            pltpu.make_async_copy(ei_hbm.at[1, pl.ds(base + i * K, K)],
                                  dstv[b], si[b]).wait()

        def gather_start(b):
            pltpu.async_copy(x_hbm.at[srcv[b]], rows[b], sg[b])

        idx_start(0, 0)
        idx_start(1, 1)
        idx_wait(0, 0)
        gather_start(0)

        def stage(i, b):
            @pl.when(i + 1 < CH)
            def _():
                idx_wait(i + 1, 1 - b)
                gather_start(1 - b)

            pltpu.make_async_copy(x_hbm.at[srcv[b]], rows[b], sg[b]).wait()
            pltpu.async_copy(rows[b], agg_sh.at[dstv[b]], ss[b], add=True)
            pltpu.make_async_copy(rows[b], agg_sh.at[dstv[b]], ss[b]).wait()

            @pl.when(i + 2 < CH)
            def _():
                idx_start(i + 2, b)

        def pair(g, carry):
            stage(2 * g, 0)
            stage(2 * g + 1, 1)
            return carry

        lax.fori_loop(0, CH // 2, pair, 0)

        toff = base + CH * K
        rowst = rows0.at[pl.ds(0, KT)]
        pltpu.sync_copy(ei_hbm.at[0, pl.ds(toff, KT)], srct)
        pltpu.sync_copy(ei_hbm.at[1, pl.ds(toff, KT)], dstt)
        pltpu.async_copy(x_hbm.at[srct], rowst, sg0).wait()
        pltpu.async_copy(rowst, agg_sh.at[dstt], ss0, add=True)
        pltpu.make_async_copy(rowst, agg_sh.at[dstt], ss0).wait()
        plsc.subcore_barrier()

        for j, sz in ((0, K), (1, K), (2, K), (3, K), (4, ROWS_PER_TILE - 4 * K)):
            pltpu.sync_copy(agg_sh.at[pl.ds(r0 + j * K, sz)],
                            rows0.at[pl.ds(0, sz)])
            pltpu.sync_copy(rows0.at[pl.ds(0, sz)],
                            agg_out.at[cid, pl.ds(r0 + j * K, sz)])

    return sc_body(x, edge_index, zeros_big)


def _sc_se(edge_index, ea128, zeros_se):
    mesh = plsc.VectorSubcoreMesh(core_axis_name="c", subcore_axis_name="s")

    @functools.partial(
        pl.kernel,
        out_type=jax.ShapeDtypeStruct((NC, N_PAD, DE), jnp.float32),
        mesh=mesh,
        scratch_types=[
            pltpu.VMEM((K,), jnp.int32),
            pltpu.VMEM((K,), jnp.int32),
            pltpu.VMEM((K,), jnp.int32),
            pltpu.VMEM((K,), jnp.int32),
            pltpu.VMEM((K,), jnp.int32),
            pltpu.VMEM((K,), jnp.int32),
            pltpu.VMEM((K,), jnp.int32),
            pltpu.VMEM((K,), jnp.int32),
            pltpu.VMEM((Q * K * DE // 128, 128), jnp.float32),
            pltpu.VMEM((Q * K * DE // 128, 128), jnp.float32),
            pltpu.VMEM((K, DE), jnp.float32),
            pltpu.VMEM((K, DE), jnp.float32),
            pltpu.VMEM((K, DE), jnp.float32),
            pltpu.VMEM((K, DE), jnp.float32),
            pltpu.VMEM_SHARED((N_PAD, DE), jnp.float32),
            pltpu.SemaphoreType.DMA,
            pltpu.SemaphoreType.DMA,
            pltpu.SemaphoreType.DMA,
        ],
        compiler_params=pltpu.CompilerParams(use_tc_tiling_on_sc=False),
    )
    def sc_body(ei_hbm, ea_hbm, zse_hbm, se_out,
                d00, d01, d02, d03, d10, d11, d12, d13,
                ear0, ear1, e0, e1, e2, e3,
                se_sh, sl0, sl1, ss):
        cid = lax.axis_index("c")
        sid = lax.axis_index("s")
        wid = cid * NS + sid
        base = wid * EPT_B
        nst = jnp.where(wid == NW - 1, SB_LAST, SB_FULL)
        dv = ((d00, d01, d02, d03), (d10, d11, d12, d13))
        ear = (ear0, ear1)
        sl = (sl0, sl1)
        eb = (e0, e1, e2, e3)
        ER = Q * K * DE // 128

        r0 = sid * ROWS_PER_TILE
        pltpu.sync_copy(zse_hbm, e0)
        for j, sz in ((0, K), (1, K), (2, K), (3, K), (4, ROWS_PER_TILE - 4 * K)):
            pltpu.sync_copy(e0.at[pl.ds(0, sz)],
                            se_sh.at[pl.ds(r0 + j * K, sz)])
        plsc.subcore_barrier()

        def loads_start(i, b):
            off = base + i * Q * K
            for j in range(Q):
                pltpu.async_copy(ei_hbm.at[1, pl.ds(off + j * K, K)],
                                 dv[b][j], sl[b])
            pltpu.async_copy(ea_hbm.at[pl.ds(off * DE // 128, ER)], ear[b], sl[b])

        def loads_wait(i, b):
            off = base + i * Q * K
            for j in range(Q):
                pltpu.make_async_copy(ei_hbm.at[1, pl.ds(off + j * K, K)],
                                      dv[b][j], sl[b]).wait()
            pltpu.make_async_copy(ea_hbm.at[pl.ds(off * DE // 128, ER)],
                                  ear[b], sl[b]).wait()

        loads_start(0, 0)

        def stage(i, b):
            loads_wait(i, b)

            @pl.when(i + 1 < nst)
            def _():
                loads_start(i + 1, 1 - b)

            for j in range(Q):
                for q in range(K):
                    r = j * (K * DE // 128) + q // 8
                    eb[j][q, :] = ear[b][r, pl.ds((q % 8) * DE, DE)]
            for j in range(Q):
                pltpu.async_copy(eb[j], se_sh.at[dv[b][j]], ss, add=True)
            for j in range(Q):
                pltpu.make_async_copy(eb[j], se_sh.at[dv[b][j]], ss).wait()

        def pair(g, carry):
            @pl.when(2 * g < nst)
            def _():
                stage(2 * g, 0)

            @pl.when(2 * g + 1 < nst)
            def _():
                stage(2 * g + 1, 1)
            return carry

        lax.fori_loop(0, (SB_FULL + 1) // 2, pair, 0)
        plsc.subcore_barrier()

        for j, sz in ((0, K), (1, K), (2, K), (3, K), (4, ROWS_PER_TILE - 4 * K)):
            pltpu.sync_copy(se_sh.at[pl.ds(r0 + j * K, sz)],
                            e0.at[pl.ds(0, sz)])
            pltpu.sync_copy(e0.at[pl.ds(0, sz)],
                            se_out.at[cid, pl.ds(r0 + j * K, sz)])

    return sc_body(edge_index, ea128, zeros_se)


def _tc_body(aggp_ref, sep_ref, x_ref, We_ref, Wr_ref, br_ref, Wo_ref,
             g_ref, be_ref, out_ref):
    agg = aggp_ref[0, :N, :] + aggp_ref[1, :N, :]
    se = sep_ref[0, :N, :] + sep_ref[1, :N, :]
    x = x_ref[...]
    ea = lax.dot_general(se, We_ref[...], (((1,), (1,)), ((), ())),
                         preferred_element_type=jnp.float32)
    m = agg + ea
    pre = lax.dot_general(m, Wr_ref[...], (((1,), (1,)), ((), ())),
                          preferred_element_type=jnp.float32)
    pre = pre + lax.dot_general(x, Wo_ref[...], (((1,), (1,)), ((), ())),
                                preferred_element_type=jnp.float32)
    pre = pre + br_ref[...]
    pre = jnp.maximum(pre, 0.0)
    mean = jnp.mean(pre, axis=0, keepdims=True)
    var = jnp.mean((pre - mean) ** 2, axis=0, keepdims=True)
    out_ref[...] = (pre - mean) * lax.rsqrt(var + 1e-5) * g_ref[...] + be_ref[...]


def kernel(x, edge_index, edge_attr, W_edge, W_rel, b_rel, W_root, gamma, beta):
    ei = edge_index.astype(jnp.int32)
    ea128 = edge_attr.reshape(E * DE // 128, 128)
    zeros_big = jnp.zeros((K, D), jnp.float32)
    zeros_se = jnp.zeros((K, DE), jnp.float32)

    aggp = _sc_agg(x, ei, zeros_big)
    sep = _sc_se(ei, ea128, zeros_se)

    out = pl.pallas_call(
        _tc_body,
        out_shape=jax.ShapeDtypeStruct((N, D), jnp.float32),
    )(aggp, sep, x, W_edge, W_rel, b_rel.reshape(1, D), W_root,
      gamma.reshape(1, D), beta.reshape(1, D))
    return out


# speedup vs baseline: 1.0430x; 1.0430x over previous
import functools

import jax
import jax.numpy as jnp
from jax import lax
from jax.experimental import pallas as pl
from jax.experimental.pallas import tpu as pltpu
from jax.experimental.pallas import tpu_sc as plsc

N = 10000
E = 320000
D = 128
DE = 16

NC = 2
NS = 16
NW = NC * NS
EPT = E // NW
K = 128
CH = 78
KT = EPT - CH * K
N_PAD = 10112
ROWS_PER_TILE = N_PAD // NS


def _sc_scatter(x, edge_index, edge_attr, zeros_big, zeros_se):
    mesh = plsc.VectorSubcoreMesh(core_axis_name="c", subcore_axis_name="s")

    @functools.partial(
        pl.kernel,
        out_type=(
            jax.ShapeDtypeStruct((NC, N_PAD, D), jnp.float32),
            jax.ShapeDtypeStruct((NC, N_PAD, DE), jnp.float32),
        ),
        mesh=mesh,
        scratch_types=[
            pltpu.VMEM((K,), jnp.int32),
            pltpu.VMEM((K,), jnp.int32),
            pltpu.VMEM((K,), jnp.int32),
            pltpu.VMEM((K,), jnp.int32),
            pltpu.VMEM((KT,), jnp.int32),
            pltpu.VMEM((KT,), jnp.int32),
            pltpu.VMEM((K, D), jnp.float32),
            pltpu.VMEM((K, D), jnp.float32),
            pltpu.VMEM((KT, D), jnp.float32),
            pltpu.VMEM((K, DE), jnp.float32),
            pltpu.VMEM((K, DE), jnp.float32),
            pltpu.VMEM((KT, DE), jnp.float32),
            pltpu.VMEM_SHARED((N_PAD, D), jnp.float32),
            pltpu.VMEM_SHARED((N_PAD, DE), jnp.float32),
            pltpu.SemaphoreType.DMA,
            pltpu.SemaphoreType.DMA,
            pltpu.SemaphoreType.DMA,
            pltpu.SemaphoreType.DMA,
            pltpu.SemaphoreType.DMA,
            pltpu.SemaphoreType.DMA,
        ],
        compiler_params=pltpu.CompilerParams(use_tc_tiling_on_sc=False),
    )
    def sc_body(x_hbm, ei_hbm, ea_hbm, z_hbm, zse_hbm,
                agg_out, se_out, srcv0, srcv1, dstv0, dstv1, srct, dstt,
                rows0, rows1, rowst, eab0, eab1, eat,
                agg_sh, se_sh, si0, si1, sg0, sg1, ss0, ss1):
        cid = lax.axis_index("c")
        sid = lax.axis_index("s")
        wid = cid * NS + sid
        base = wid * EPT
        srcv = (srcv0, srcv1)
        dstv = (dstv0, dstv1)
        rows = (rows0, rows1)
        eab = (eab0, eab1)
        si = (si0, si1)
        sg = (sg0, sg1)
        ss = (ss0, ss1)

        r0 = sid * ROWS_PER_TILE
        pltpu.sync_copy(z_hbm, rows0)
        pltpu.sync_copy(zse_hbm, eab0)
        for j, sz in ((0, K), (1, K), (2, K), (3, K), (4, ROWS_PER_TILE - 4 * K)):
            pltpu.sync_copy(rows0.at[pl.ds(0, sz)],
                            agg_sh.at[pl.ds(r0 + j * K, sz)])
            pltpu.sync_copy(eab0.at[pl.ds(0, sz)],
                            se_sh.at[pl.ds(r0 + j * K, sz)])
        plsc.subcore_barrier()

        def idx_start(i, b):
            pltpu.async_copy(ei_hbm.at[0, pl.ds(base + i * K, K)], srcv[b], si[b])
            pltpu.async_copy(ei_hbm.at[1, pl.ds(base + i * K, K)], dstv[b], si[b])

        def idx_wait(i, b):
            pltpu.make_async_copy(ei_hbm.at[0, pl.ds(base + i * K, K)],
                                  srcv[b], si[b]).wait()
            pltpu.make_async_copy(ei_hbm.at[1, pl.ds(base + i * K, K)],
                                  dstv[b], si[b]).wait()

        def gather_start(i, b):
            pltpu.async_copy(x_hbm.at[srcv[b]], rows[b], sg[b])
            pltpu.async_copy(ea_hbm.at[pl.ds(base + i * K, K)], eab[b], sg[b])

        idx_start(0, 0)
        idx_start(1, 1)
        idx_wait(0, 0)
        gather_start(0, 0)

        def stage(i, b):
            @pl.when(i + 1 < CH)
            def _():
                idx_wait(i + 1, 1 - b)
                gather_start(i + 1, 1 - b)

            pltpu.make_async_copy(x_hbm.at[srcv[b]], rows[b], sg[b]).wait()
            pltpu.make_async_copy(ea_hbm.at[pl.ds(base + i * K, K)],
                                  eab[b], sg[b]).wait()
            pltpu.async_copy(rows[b], agg_sh.at[dstv[b]], ss[b], add=True)
            pltpu.async_copy(eab[b], se_sh.at[dstv[b]], ss[b], add=True)
            pltpu.make_async_copy(rows[b], agg_sh.at[dstv[b]], ss[b]).wait()
            pltpu.make_async_copy(eab[b], se_sh.at[dstv[b]], ss[b]).wait()

            @pl.when(i + 2 < CH)
            def _():
                idx_start(i + 2, b)

        def pair(g, carry):
            stage(2 * g, 0)
            stage(2 * g + 1, 1)
            return carry

        lax.fori_loop(0, CH // 2, pair, 0)

        toff = base + CH * K
        pltpu.sync_copy(ei_hbm.at[0, pl.ds(toff, KT)], srct)
        pltpu.sync_copy(ei_hbm.at[1, pl.ds(toff, KT)], dstt)
        pltpu.sync_copy(ea_hbm.at[pl.ds(toff, KT)], eat)
        pltpu.async_copy(x_hbm.at[srct], rowst, sg0).wait()
        pltpu.async_copy(rowst, agg_sh.at[dstt], ss0, add=True)
        pltpu.async_copy(eat, se_sh.at[dstt], ss0, add=True)
        pltpu.make_async_copy(rowst, agg_sh.at[dstt], ss0).wait()
        pltpu.make_async_copy(eat, se_sh.at[dstt], ss0).wait()
        plsc.subcore_barrier()

        for j, sz in ((0, K), (1, K), (2, K), (3, K), (4, ROWS_PER_TILE - 4 * K)):
            pltpu.sync_copy(agg_sh.at[pl.ds(r0 + j * K, sz)],
                            rows0.at[pl.ds(0, sz)])
            pltpu.sync_copy(rows0.at[pl.ds(0, sz)],
                            agg_out.at[cid, pl.ds(r0 + j * K, sz)])
            pltpu.sync_copy(se_sh.at[pl.ds(r0 + j * K, sz)],
                            eab0.at[pl.ds(0, sz)])
            pltpu.sync_copy(eab0.at[pl.ds(0, sz)],
                            se_out.at[cid, pl.ds(r0 + j * K, sz)])

    return sc_body(x, edge_index, edge_attr, zeros_big, zeros_se)


def _tc_body(aggp_ref, sep_ref, x_ref, We_ref, Wr_ref, br_ref, Wo_ref,
             g_ref, be_ref, out_ref):
    agg = aggp_ref[0, :N, :] + aggp_ref[1, :N, :]
    se = sep_ref[0, :N, :] + sep_ref[1, :N, :]
    x = x_ref[...]
    ea = lax.dot_general(se, We_ref[...], (((1,), (1,)), ((), ())),
                         preferred_element_type=jnp.float32)
    m = agg + ea
    pre = lax.dot_general(m, Wr_ref[...], (((1,), (1,)), ((), ())),
                          preferred_element_type=jnp.float32)
    pre = pre + lax.dot_general(x, Wo_ref[...], (((1,), (1,)), ((), ())),
                                preferred_element_type=jnp.float32)
    pre = pre + br_ref[...]
    pre = jnp.maximum(pre, 0.0)
    mean = jnp.mean(pre, axis=0, keepdims=True)
    var = jnp.mean((pre - mean) ** 2, axis=0, keepdims=True)
    out_ref[...] = (pre - mean) * lax.rsqrt(var + 1e-5) * g_ref[...] + be_ref[...]


def kernel(x, edge_index, edge_attr, W_edge, W_rel, b_rel, W_root, gamma, beta):
    ei = edge_index.astype(jnp.int32)
    zeros_big = jnp.zeros((K, D), jnp.float32)
    zeros_se = jnp.zeros((K, DE), jnp.float32)

    aggp, sep = _sc_scatter(x, ei, edge_attr, zeros_big, zeros_se)

    out = pl.pallas_call(
        _tc_body,
        out_shape=jax.ShapeDtypeStruct((N, D), jnp.float32),
    )(aggp, sep, x, W_edge, W_rel, b_rel.reshape(1, D), W_root,
      gamma.reshape(1, D), beta.reshape(1, D))
    return out
